# Initial kernel scaffold; baseline (speedup 1.0000x reference)
import jax
import jax.numpy as jnp
from jax.experimental import pallas as pl


def kernel(hidden_states, gate_w, w_gate_up, w_down, ws_gate_up, ws_down):
    raise NotImplementedError("write your pallas kernel here")


# speedup vs baseline: 1.3594x; 1.3594x over previous
import functools

import jax
import jax.numpy as jnp
from jax.experimental import pallas as pl
from jax.experimental.pallas import tpu as pltpu

T = 2048
D = 1024
FF = 512
E = 8
SHARED_FF2 = 2048


def _moe_body(x_ref, xb_ref, gate_ref, wgu_ref, wd_ref, wsg_ref, wsd_ref,
              out_ref):
    x = x_ref[...]
    xb = xb_ref[...]

    logits = jnp.dot(x, gate_ref[...].T, preferred_element_type=jnp.float32)
    probs = jax.nn.softmax(logits, axis=-1)
    iota_e = jax.lax.broadcasted_iota(jnp.int32, probs.shape, 1)
    m1 = jnp.max(probs, axis=1, keepdims=True)
    idx1 = jnp.min(jnp.where(probs == m1, iota_e, E), axis=1, keepdims=True)
    oh1 = (iota_e == idx1)
    masked = jnp.where(oh1, -1.0, probs)
    m2 = jnp.max(masked, axis=1, keepdims=True)
    idx2 = jnp.min(jnp.where(masked == m2, iota_e, E), axis=1, keepdims=True)
    oh2 = (iota_e == idx2)
    wsum = m1 + m2
    combine = jnp.where(oh1, m1 / wsum, 0.0) + jnp.where(oh2, m2 / wsum, 0.0)

    acc = jnp.zeros(out_ref.shape, jnp.float32)
    for e in range(E):
        gu = jnp.dot(xb, wgu_ref[e].T, preferred_element_type=jnp.float32)
        g, u = gu[:, :FF], gu[:, FF:]
        h = (g * jax.nn.sigmoid(g) * u).astype(jnp.bfloat16)
        oe = jnp.dot(h, wd_ref[e].T, preferred_element_type=jnp.float32)
        acc = acc + combine[:, e:e + 1] * oe

    sgu = jnp.dot(xb, wsg_ref[...].T, preferred_element_type=jnp.float32)
    sg, su = sgu[:, :SHARED_FF2 // 2], sgu[:, SHARED_FF2 // 2:]
    sh = (sg * jax.nn.sigmoid(sg) * su).astype(jnp.bfloat16)
    acc = acc + jnp.dot(sh, wsd_ref[...].T, preferred_element_type=jnp.float32)

    out_ref[...] = acc


def kernel(hidden_states, gate_w, w_gate_up, w_down, ws_gate_up, ws_down):
    b, s, d = hidden_states.shape
    x = hidden_states.reshape(-1, d)
    xb = x.astype(jnp.bfloat16)
    wgu = w_gate_up.astype(jnp.bfloat16)
    wd = w_down.astype(jnp.bfloat16)
    wsg = ws_gate_up.astype(jnp.bfloat16)
    wsd = ws_down.astype(jnp.bfloat16)

    TM = 256
    grid = (T // TM,)
    out = pl.pallas_call(
        _moe_body,
        grid=grid,
        in_specs=[
            pl.BlockSpec((TM, D), lambda i: (i, 0)),
            pl.BlockSpec((TM, D), lambda i: (i, 0)),
            pl.BlockSpec((E, D), lambda i: (0, 0)),
            pl.BlockSpec((E, 2 * FF, D), lambda i: (0, 0, 0)),
            pl.BlockSpec((E, D, FF), lambda i: (0, 0, 0)),
            pl.BlockSpec((SHARED_FF2, D), lambda i: (0, 0)),
            pl.BlockSpec((D, SHARED_FF2 // 2), lambda i: (0, 0)),
        ],
        out_specs=pl.BlockSpec((TM, D), lambda i: (i, 0)),
        out_shape=jax.ShapeDtypeStruct((T, D), jnp.float32),
    )(x, xb, gate_w, wgu, wd, wsg, wsd)
    return out.reshape(b, s, d)
